# baseline (speedup 1.0000x reference)
import functools

import jax
import jax.numpy as jnp
from jax import lax
from jax.experimental import pallas as pl
from jax.experimental.pallas import tpu as pltpu
from jax.experimental.pallas import tpu_sc as plsc

_SUB = 128
_NSUB = 4
_CHUNK = _SUB * _NSUB
_NBUF = 2
_LANES = 16


@functools.lru_cache(maxsize=None)
def _build(vocab, emb, n_tokens):
    info = plsc.get_sparse_core_info()
    nc, ns = info.num_cores, info.num_subcores
    nw = nc * ns
    assert n_tokens % (nw * _CHUNK) == 0
    per_w = n_tokens // nw
    n_chunks = per_w // _CHUNK
    assert n_chunks % _NBUF == 0
    idx_rows_per_w = per_w // _SUB
    scale = float(emb) ** 0.5
    mesh = plsc.VectorSubcoreMesh(core_axis_name="c", subcore_axis_name="s")

    @functools.partial(
        pl.kernel,
        out_type=jax.ShapeDtypeStruct((n_tokens, emb), jnp.float32),
        mesh=mesh,
        compiler_params=pltpu.CompilerParams(use_tc_tiling_on_sc=False),
        scratch_types=[
            pltpu.VMEM((_NSUB, _SUB), jnp.int32),
            pltpu.VMEM((_NSUB, _SUB), jnp.int32),
            pltpu.VMEM((_CHUNK, emb), jnp.float32),
            pltpu.VMEM((_CHUNK, emb), jnp.float32),
            pltpu.SemaphoreType.DMA,
            pltpu.SemaphoreType.DMA,
            pltpu.SemaphoreType.DMA,
            pltpu.SemaphoreType.DMA,
        ],
    )
    def emb_kernel(table_hbm, idx_hbm, out_hbm,
                   idx0, idx1, rows0, rows1, gsem0, gsem1, osem0, osem1):
        wid = lax.axis_index("s") * nc + lax.axis_index("c")
        idx_base = wid * idx_rows_per_w
        row_base = wid * per_w
        bufs = ((idx0, rows0, gsem0, osem0), (idx1, rows1, gsem1, osem1))

        def issue_chunk(b, c):
            idx_v, rows_v, gsem, _ = bufs[b]
            pltpu.sync_copy(idx_hbm.at[pl.ds(idx_base + c * _NSUB, _NSUB)],
                            idx_v)
            for k in range(_NSUB):
                pltpu.async_copy(table_hbm.at[idx_v.at[k]],
                                 rows_v.at[pl.ds(k * _SUB, _SUB)], gsem)

        def wait_gathers(b):
            idx_v, rows_v, gsem, _ = bufs[b]
            for k in range(_NSUB):
                pltpu.make_async_copy(table_hbm.at[idx_v.at[k]],
                                      rows_v.at[pl.ds(k * _SUB, _SUB)],
                                      gsem).wait()

        def out_slice(c):
            return out_hbm.at[pl.ds(row_base + c * _CHUNK, _CHUNK)]

        for b in range(_NBUF):
            issue_chunk(b, b)

        @pl.loop(0, n_chunks, step=_NBUF)
        def _chunk_loop(g):
            for b in range(_NBUF):
                c = g + b
                idx_v, rows_v, _, osem = bufs[b]
                wait_gathers(b)

                @pl.loop(0, _CHUNK)
                def _scale(i):
                    for j in range(emb // _LANES):
                        sl = pl.ds(j * _LANES, _LANES)
                        rows_v[i, sl] = rows_v[i, sl] * scale

                pltpu.async_copy(rows_v, out_slice(c), osem)

                @pl.when(c + _NBUF < n_chunks)
                def _prefetch():
                    pltpu.make_async_copy(rows_v, out_slice(c), osem).wait()
                    issue_chunk(b, c + _NBUF)

        for b in range(_NBUF):
            _, rows_v, _, osem = bufs[b]
            pltpu.make_async_copy(rows_v, out_slice(0), osem).wait()

    return emb_kernel


def kernel(token_ids, table):
    bsz, seq = token_ids.shape
    vocab, emb = table.shape
    n_tokens = bsz * seq
    idx = token_ids.reshape(n_tokens // _SUB, _SUB).astype(jnp.int32)
    out = _build(vocab, emb, n_tokens)(table, idx)
    return out.reshape(bsz, seq, emb)


# speedup vs baseline: 1.2536x; 1.2536x over previous
import functools

import jax
import jax.numpy as jnp
from jax import lax
from jax.experimental import pallas as pl
from jax.experimental.pallas import tpu as pltpu
from jax.experimental.pallas import tpu_sc as plsc

_SUB = 128
_NSUB = 2
_CHUNK = _SUB * _NSUB
_NBUF = 2
_LANES = 16
_PADW = 128


@functools.lru_cache(maxsize=None)
def _build(vocab, emb, n_tokens):
    info = plsc.get_sparse_core_info()
    nc, ns = info.num_cores, info.num_subcores
    nw = nc * ns
    assert n_tokens % (nw * _CHUNK) == 0
    per_w = n_tokens // nw
    n_chunks = per_w // _CHUNK
    assert n_chunks % _NBUF == 0
    idx_rows_per_w = per_w // _SUB
    scale = float(emb) ** 0.5
    mesh = plsc.VectorSubcoreMesh(core_axis_name="c", subcore_axis_name="s")

    @functools.partial(
        pl.kernel,
        out_type=jax.ShapeDtypeStruct((n_tokens, _PADW), jnp.float32),
        mesh=mesh,
        scratch_types=[
            pltpu.VMEM((_NSUB, _SUB), jnp.int32),
            pltpu.VMEM((_NSUB, _SUB), jnp.int32),
            pltpu.VMEM((_CHUNK, _PADW), jnp.float32),
            pltpu.VMEM((_CHUNK, _PADW), jnp.float32),
            pltpu.SemaphoreType.DMA,
            pltpu.SemaphoreType.DMA,
            pltpu.SemaphoreType.DMA,
            pltpu.SemaphoreType.DMA,
        ],
    )
    def emb_kernel(table_hbm, idx_hbm, out_hbm,
                   idx0, idx1, rows0, rows1, gsem0, gsem1, osem0, osem1):
        wid = lax.axis_index("s") * nc + lax.axis_index("c")
        idx_base = wid * idx_rows_per_w
        row_base = wid * per_w
        bufs = ((idx0, rows0, gsem0, osem0), (idx1, rows1, gsem1, osem1))

        def issue_chunk(b, c):
            idx_v, rows_v, gsem, _ = bufs[b]
            pltpu.sync_copy(idx_hbm.at[pl.ds(idx_base + c * _NSUB, _NSUB)],
                            idx_v)
            for k in range(_NSUB):
                pltpu.async_copy(table_hbm.at[idx_v.at[k]],
                                 rows_v.at[pl.ds(k * _SUB, _SUB)], gsem)

        def wait_gathers(b):
            idx_v, rows_v, gsem, _ = bufs[b]
            for k in range(_NSUB):
                pltpu.make_async_copy(table_hbm.at[idx_v.at[k]],
                                      rows_v.at[pl.ds(k * _SUB, _SUB)],
                                      gsem).wait()

        def out_copy(b, c):
            _, rows_v, _, osem = bufs[b]
            return pltpu.make_async_copy(
                rows_v,
                out_hbm.at[pl.ds(row_base + c * _CHUNK, _CHUNK)], osem)

        for b in range(_NBUF):
            issue_chunk(b, b)

        @pl.loop(0, n_chunks, step=_NBUF)
        def _chunk_loop(g):
            for b in range(_NBUF):
                c = g + b
                idx_v, rows_v, _, osem = bufs[b]
                wait_gathers(b)

                @pl.loop(0, _CHUNK)
                def _scale(i):
                    for j in range(emb // _LANES):
                        sl = pl.ds(j * _LANES, _LANES)
                        rows_v[i, sl] = rows_v[i, sl] * scale

                out_copy(b, c).start()

                @pl.when(c + _NBUF < n_chunks)
                def _prefetch():
                    out_copy(b, c).wait()
                    issue_chunk(b, c + _NBUF)

        for b in range(_NBUF):
            out_copy(b, n_chunks - _NBUF + b).wait()

    return emb_kernel


def kernel(token_ids, table):
    bsz, seq = token_ids.shape
    vocab, emb = table.shape
    n_tokens = bsz * seq
    table_pad = jnp.pad(table, ((0, 0), (0, _PADW - emb)))
    idx = token_ids.reshape(n_tokens // _SUB, _SUB).astype(jnp.int32)
    out = _build(vocab, emb, n_tokens)(table_pad, idx)
    return out[:, :emb].reshape(bsz, seq, emb)


# speedup vs baseline: 1.3679x; 1.0912x over previous
import functools

import jax
import jax.numpy as jnp
from jax import lax
from jax.experimental import pallas as pl
from jax.experimental.pallas import tpu as pltpu
from jax.experimental.pallas import tpu_sc as plsc

_SUB = 128
_NSUB = 2
_CHUNK = _SUB * _NSUB
_NBUF = 2
_PADW = 128
_BV = 2048


def _relayout_block(t_ref, o_ref, *, scale):
    o_ref[:, 0:t_ref.shape[0]] = t_ref[...].T * scale


@functools.lru_cache(maxsize=None)
def _build_relayout(vocab, emb):
    grid = (vocab + _BV - 1) // _BV
    return pl.pallas_call(
        functools.partial(_relayout_block, scale=float(emb) ** 0.5),
        grid=(grid,),
        in_specs=[pl.BlockSpec((emb, _BV), lambda i: (0, i))],
        out_specs=pl.BlockSpec((_BV, _PADW), lambda i: (i, 0)),
        out_shape=jax.ShapeDtypeStruct((vocab, _PADW), jnp.float32),
    )


@functools.lru_cache(maxsize=None)
def _build_gather(vocab, emb, n_tokens):
    info = plsc.get_sparse_core_info()
    nc, ns = info.num_cores, info.num_subcores
    nw = nc * ns
    assert n_tokens % (nw * _CHUNK) == 0
    per_w = n_tokens // nw
    n_chunks = per_w // _CHUNK
    assert n_chunks % _NBUF == 0
    idx_rows_per_w = per_w // _SUB
    mesh = plsc.VectorSubcoreMesh(core_axis_name="c", subcore_axis_name="s")

    @functools.partial(
        pl.kernel,
        out_type=jax.ShapeDtypeStruct((n_tokens, _PADW), jnp.float32),
        mesh=mesh,
        scratch_types=[
            pltpu.VMEM((_NSUB, _SUB), jnp.int32),
            pltpu.VMEM((_NSUB, _SUB), jnp.int32),
            pltpu.VMEM((_CHUNK, _PADW), jnp.float32),
            pltpu.VMEM((_CHUNK, _PADW), jnp.float32),
            pltpu.SemaphoreType.DMA,
            pltpu.SemaphoreType.DMA,
            pltpu.SemaphoreType.DMA,
            pltpu.SemaphoreType.DMA,
        ],
    )
    def emb_kernel(table_hbm, idx_hbm, out_hbm,
                   idx0, idx1, rows0, rows1, gsem0, gsem1, osem0, osem1):
        wid = lax.axis_index("s") * nc + lax.axis_index("c")
        idx_base = wid * idx_rows_per_w
        row_base = wid * per_w
        bufs = ((idx0, rows0, gsem0, osem0), (idx1, rows1, gsem1, osem1))

        def issue_chunk(b, c):
            idx_v, rows_v, gsem, _ = bufs[b]
            pltpu.sync_copy(idx_hbm.at[pl.ds(idx_base + c * _NSUB, _NSUB)],
                            idx_v)
            for k in range(_NSUB):
                pltpu.async_copy(table_hbm.at[idx_v.at[k]],
                                 rows_v.at[pl.ds(k * _SUB, _SUB)], gsem)

        def wait_gathers(b):
            idx_v, rows_v, gsem, _ = bufs[b]
            for k in range(_NSUB):
                pltpu.make_async_copy(table_hbm.at[idx_v.at[k]],
                                      rows_v.at[pl.ds(k * _SUB, _SUB)],
                                      gsem).wait()

        def out_copy(b, c):
            _, rows_v, _, osem = bufs[b]
            return pltpu.make_async_copy(
                rows_v,
                out_hbm.at[pl.ds(row_base + c * _CHUNK, _CHUNK)], osem)

        for b in range(_NBUF):
            issue_chunk(b, b)

        @pl.loop(0, n_chunks, step=_NBUF)
        def _chunk_loop(g):
            for b in range(_NBUF):
                c = g + b
                wait_gathers(b)
                out_copy(b, c).start()

                @pl.when(c + _NBUF < n_chunks)
                def _prefetch():
                    out_copy(b, c).wait()
                    issue_chunk(b, c + _NBUF)

        for b in range(_NBUF):
            out_copy(b, n_chunks - _NBUF + b).wait()

    return emb_kernel


def kernel(token_ids, table):
    bsz, seq = token_ids.shape
    vocab, emb = table.shape
    n_tokens = bsz * seq
    table_pad = _build_relayout(vocab, emb)(table.T)
    idx = token_ids.reshape(n_tokens // _SUB, _SUB).astype(jnp.int32)
    out = _build_gather(vocab, emb, n_tokens)(table_pad, idx)
    return out[:, :emb].reshape(bsz, seq, emb)
